# baseline (speedup 1.0000x reference)
import functools

import jax
import jax.numpy as jnp
from jax import lax
from jax.experimental import pallas as pl
from jax.experimental.pallas import tpu as pltpu
from jax.experimental.pallas import tpu_sc as plsc

N = 10000
E = 320000
F = 128
FCLS = 40

NSC = 2
NSUB = 16
NW = NSC * NSUB
NPAD = 10240
RPS = NPAD // NSUB
EPW = E // NW
K = 200

_sc_mesh = plsc.VectorSubcoreMesh(
    core_axis_name="c", subcore_axis_name="s", num_cores=NSC, num_subcores=NSUB
)


EPW2 = E // NSUB


@functools.partial(
    pl.kernel,
    out_type=jax.ShapeDtypeStruct((NPAD,), jnp.float32),
    mesh=_sc_mesh,
    scratch_types=[
        pltpu.VMEM((EPW2,), jnp.int32),
        pltpu.VMEM((EPW2,), jnp.float32),
        pltpu.VMEM_SHARED((NPAD,), jnp.float32),
        pltpu.SemaphoreType.DMA,
        pltpu.SemaphoreType.DMA,
    ],
)
def _sc_degree(dst_hbm, z_hbm, ones_hbm, out_hbm, idx_v, ones_v, deg_sh, s1, s2):
    cc = lax.axis_index("c")
    sid = lax.axis_index("s")
    r0 = sid * RPS
    base = sid * EPW2
    pltpu.async_copy(dst_hbm.at[pl.ds(base, EPW2)], idx_v, s1)
    pltpu.async_copy(ones_hbm, ones_v, s2)
    pltpu.sync_copy(z_hbm.at[pl.ds(r0, RPS)], deg_sh.at[pl.ds(r0, RPS)])
    pltpu.make_async_copy(dst_hbm.at[pl.ds(base, EPW2)], idx_v, s1).wait()
    pltpu.make_async_copy(ones_hbm, ones_v, s2).wait()
    plsc.subcore_barrier()

    pltpu.sync_copy(ones_v, deg_sh.at[idx_v], add=True)

    plsc.subcore_barrier()

    @pl.when(cc == 0)
    def _():
        pltpu.sync_copy(deg_sh.at[pl.ds(r0, RPS)], out_hbm.at[pl.ds(r0, RPS)])


KC = 80
NCHUNK = EPW // KC
NSLOT = 3


@functools.partial(
    pl.kernel,
    out_type=jax.ShapeDtypeStruct((NSC, NPAD, F), jnp.float32),
    mesh=_sc_mesh,
    scratch_types=[
        pltpu.VMEM((EPW,), jnp.int32),
        pltpu.VMEM((NSLOT, KC), jnp.int32),
        pltpu.VMEM((NSLOT, KC, F), jnp.float32),
        pltpu.VMEM_SHARED((NPAD, F), jnp.float32),
        pltpu.SemaphoreType.DMA((NSLOT,)),
        pltpu.SemaphoreType.DMA((NSLOT,)),
        pltpu.SemaphoreType.DMA((NSLOT,)),
        pltpu.SemaphoreType.DMA,
    ],
)
def _sc_aggregate(g_hbm, src_hbm, dst_hbm, z_hbm, out_hbm,
                  srcall, didx, rows_v, acc_sh, sem_i, sem_g, sem_s, sem_m):
    cc = lax.axis_index("c")
    sid = lax.axis_index("s")
    r0 = sid * RPS
    base = (cc * NSUB + sid) * EPW

    pltpu.async_copy(src_hbm.at[pl.ds(base, EPW)], srcall, sem_m)
    pltpu.sync_copy(z_hbm.at[pl.ds(r0, RPS)], acc_sh.at[pl.ds(r0, RPS)])
    pltpu.make_async_copy(src_hbm.at[pl.ds(base, EPW)], srcall, sem_m).wait()
    plsc.subcore_barrier()

    def issue_front(jj, slot):
        pltpu.async_copy(dst_hbm.at[pl.ds(base + jj * KC, KC)], didx.at[slot],
                         sem_i.at[slot])
        pltpu.async_copy(g_hbm.at[srcall.at[pl.ds(jj * KC, KC)]], rows_v.at[slot],
                         sem_g.at[slot])

    def wait_front(slot):
        pltpu.make_async_copy(dst_hbm.at[pl.ds(0, KC)], didx.at[slot],
                              sem_i.at[slot]).wait()
        pltpu.make_async_copy(g_hbm.at[pl.ds(0, KC)], rows_v.at[slot],
                              sem_g.at[slot]).wait()

    def issue_scatter(slot):
        pass

    def wait_scatter(slot):
        pass

    issue_front(0, 0)
    issue_front(1, 1)
    wait_front(0)
    issue_scatter(0)
    issue_front(2, 2)
    wait_front(1)
    issue_scatter(1)
    wait_scatter(0)
    issue_front(3, 0)
    wait_front(2)
    issue_scatter(2)
    wait_scatter(1)
    issue_front(4, 1)

    @pl.loop(3, NCHUNK - 2, step=NSLOT)
    def _(c):
        for slot_off in range(NSLOT):
            jj = c + slot_off
            slot = (3 + slot_off) % NSLOT
            wait_front(slot)
            issue_scatter(slot)
            nslot = (slot + 2) % NSLOT
            wait_scatter(nslot)
            issue_front(jj + 2, nslot)

    wait_front(0)
    issue_scatter(0)
    wait_front(1)
    issue_scatter(1)
    wait_scatter(2)
    wait_scatter(0)
    wait_scatter(1)

    plsc.subcore_barrier()
    pltpu.sync_copy(acc_sh.at[pl.ds(r0, RPS)], out_hbm.at[cc, pl.ds(r0, RPS)])


_R = 1000


def _tc_matmul_body(x_ref, w_ref, h_ref):
    h_ref[...] = jnp.dot(x_ref[...], w_ref[...],
                         preferred_element_type=jnp.float32)


def _tc_matmul(x, W1):
    return pl.pallas_call(
        _tc_matmul_body,
        grid=(N // _R,),
        in_specs=[
            pl.BlockSpec((_R, F), lambda i: (i, 0)),
            pl.BlockSpec((F, F), lambda i: (0, 0)),
        ],
        out_specs=pl.BlockSpec((_R, F), lambda i: (i, 0)),
        out_shape=jax.ShapeDtypeStruct((N, F), jnp.float32),
    )(x, W1)


def _tc_scale_body(h_ref, c_ref, g_ref):
    g_ref[...] = h_ref[...] * lax.rsqrt(c_ref[...] + 1.0)


def _tc_scale(h, cnt):
    return pl.pallas_call(
        _tc_scale_body,
        grid=(N // _R,),
        in_specs=[
            pl.BlockSpec((_R, F), lambda i: (i, 0)),
            pl.BlockSpec((_R, 1), lambda i: (i, 0)),
        ],
        out_specs=pl.BlockSpec((_R, F), lambda i: (i, 0)),
        out_shape=jax.ShapeDtypeStruct((N, F), jnp.float32),
    )(h, cnt)


def _tc_finish_body(a0_ref, a1_ref, g_ref, c_ref, b1_ref, wc_ref,
                    bc_ref, emb_ref, lsm_ref):
    dinv = lax.rsqrt(c_ref[...] + 1.0)
    s = (a0_ref[0] + a1_ref[0] + g_ref[...]) * dinv + b1_ref[...]
    emb = jnp.maximum(s, 0.0)
    emb_ref[...] = emb
    logits = jnp.dot(emb, wc_ref[...], preferred_element_type=jnp.float32)
    logits = logits + bc_ref[...]
    col = lax.broadcasted_iota(jnp.int32, (_R, F), 1)
    valid = col < FCLS
    lm = jnp.where(valid, logits, jnp.float32(-1e30))
    m = jnp.max(lm, axis=1, keepdims=True)
    ez = jnp.where(valid, jnp.exp(lm - m), 0.0)
    lsm = lm - m - jnp.log(jnp.sum(ez, axis=1, keepdims=True))
    lsm_ref[...] = lsm[:, :FCLS]


def _tc_finish(acc, g, cnt, b1r, wcp, bcp):
    return pl.pallas_call(
        _tc_finish_body,
        grid=(N // _R,),
        in_specs=[
            pl.BlockSpec((1, _R, F), lambda i: (0, i, 0)),
            pl.BlockSpec((1, _R, F), lambda i: (1, i, 0)),
            pl.BlockSpec((_R, F), lambda i: (i, 0)),
            pl.BlockSpec((_R, 1), lambda i: (i, 0)),
            pl.BlockSpec((1, F), lambda i: (0, 0)),
            pl.BlockSpec((F, F), lambda i: (0, 0)),
            pl.BlockSpec((1, F), lambda i: (0, 0)),
        ],
        out_specs=[
            pl.BlockSpec((_R, F), lambda i: (i, 0)),
            pl.BlockSpec((_R, FCLS), lambda i: (i, 0)),
        ],
        out_shape=[
            jax.ShapeDtypeStruct((N, F), jnp.float32),
            jax.ShapeDtypeStruct((N, FCLS), jnp.float32),
        ],
    )(acc, acc, g, cnt, b1r, wcp, bcp)


def kernel(x, adj, W1, b1, Wc, bc):
    src = adj[0]
    dst = adj[1]

    zeros128 = jnp.zeros((NPAD, F), jnp.float32)
    zeros1d = jnp.zeros((NPAD,), jnp.float32)
    ones1d = jnp.ones((EPW2,), jnp.float32)

    h = _tc_matmul(x, W1)
    cnt = _sc_degree(dst, zeros1d, ones1d).reshape(NPAD, 1)
    g = _tc_scale(h, cnt)

    acc = _sc_aggregate(g, src, dst, zeros128)

    b1r = b1.reshape(1, F)
    wcp = jnp.pad(Wc, ((0, 0), (0, F - FCLS)))
    bcp = jnp.pad(bc, (0, F - FCLS)).reshape(1, F)

    emb, lsm = _tc_finish(acc, g, cnt, b1r, wcp, bcp)
    return (emb, lsm)


# speedup vs baseline: 1.0771x; 1.0771x over previous
import functools

import jax
import jax.numpy as jnp
from jax import lax
from jax.experimental import pallas as pl
from jax.experimental.pallas import tpu as pltpu
from jax.experimental.pallas import tpu_sc as plsc

N = 10000
E = 320000
F = 128
FCLS = 40

NSC = 2
NSUB = 16
NW = NSC * NSUB
NPAD = 10240
RPS = NPAD // NSUB
EPW = E // NW
K = 200

_sc_mesh = plsc.VectorSubcoreMesh(
    core_axis_name="c", subcore_axis_name="s", num_cores=NSC, num_subcores=NSUB
)


EPW2 = E // NSUB


@functools.partial(
    pl.kernel,
    out_type=jax.ShapeDtypeStruct((NPAD,), jnp.float32),
    mesh=_sc_mesh,
    scratch_types=[
        pltpu.VMEM((EPW2,), jnp.int32),
        pltpu.VMEM((EPW2,), jnp.float32),
        pltpu.VMEM_SHARED((NPAD,), jnp.float32),
        pltpu.SemaphoreType.DMA,
        pltpu.SemaphoreType.DMA,
    ],
)
def _sc_degree(adj_hbm, z_hbm, ones_hbm, out_hbm, idx_v, ones_v, deg_sh, s1, s2):
    cc = lax.axis_index("c")
    sid = lax.axis_index("s")
    r0 = sid * RPS
    base = sid * EPW2
    pltpu.async_copy(adj_hbm.at[pl.ds(E + base, EPW2)], idx_v, s1)
    pltpu.async_copy(ones_hbm, ones_v, s2)
    pltpu.sync_copy(z_hbm.at[pl.ds(r0, RPS)], deg_sh.at[pl.ds(r0, RPS)])
    pltpu.make_async_copy(adj_hbm.at[pl.ds(E + base, EPW2)], idx_v, s1).wait()
    pltpu.make_async_copy(ones_hbm, ones_v, s2).wait()
    plsc.subcore_barrier()

    pltpu.sync_copy(ones_v, deg_sh.at[idx_v], add=True)

    plsc.subcore_barrier()

    @pl.when(cc == 0)
    def _():
        pltpu.sync_copy(deg_sh.at[pl.ds(r0, RPS)], out_hbm.at[pl.ds(r0, RPS)])


KC = 80
NCHUNK = EPW // KC
NSLOT = 3


@functools.partial(
    pl.kernel,
    out_type=jax.ShapeDtypeStruct((NSC, NPAD, F), jnp.float32),
    mesh=_sc_mesh,
    scratch_types=[
        pltpu.VMEM((EPW,), jnp.int32),
        pltpu.VMEM((NSLOT, 1, KC), jnp.int32),
        pltpu.VMEM((NSLOT, KC, F), jnp.float32),
        pltpu.VMEM_SHARED((NPAD, F), jnp.float32),
        pltpu.SemaphoreType.DMA((NSLOT,)),
        pltpu.SemaphoreType.DMA((NSLOT,)),
        pltpu.SemaphoreType.DMA((NSLOT,)),
        pltpu.SemaphoreType.DMA,
    ],
)
def _sc_aggregate(g_hbm, adj_hbm, z_hbm, out_hbm,
                  srcall, didx, rows_v, acc_sh, sem_i, sem_g, sem_s, sem_m):
    cc = lax.axis_index("c")
    sid = lax.axis_index("s")
    r0 = sid * RPS
    base = (cc * NSUB + sid) * EPW

    pltpu.async_copy(adj_hbm.at[pl.ds(base, EPW)], srcall, sem_m)
    pltpu.sync_copy(z_hbm.at[pl.ds(r0, RPS)], acc_sh.at[pl.ds(r0, RPS)])
    pltpu.make_async_copy(adj_hbm.at[pl.ds(base, EPW)], srcall, sem_m).wait()
    plsc.subcore_barrier()

    def issue_front(jj, slot):
        pltpu.async_copy(adj_hbm.at[pl.ds(E + base + jj * KC, KC)],
                         didx.at[slot, 0], sem_i.at[slot])
        pltpu.async_copy(g_hbm.at[srcall.at[pl.ds(jj * KC, KC)]],
                         rows_v.at[slot], sem_g.at[slot])

    def wait_front(slot):
        pltpu.make_async_copy(adj_hbm.at[pl.ds(0, KC)], didx.at[slot, 0],
                              sem_i.at[slot]).wait()
        pltpu.make_async_copy(g_hbm.at[pl.ds(0, KC)], rows_v.at[slot],
                              sem_g.at[slot]).wait()

    def issue_scatter(slot):
        pltpu.async_copy(rows_v.at[slot], acc_sh.at[didx.at[slot, 0]],
                         sem_s.at[slot], add=True)

    def wait_scatter(slot):
        pltpu.make_async_copy(g_hbm.at[pl.ds(0, KC)], rows_v.at[slot],
                              sem_s.at[slot]).wait()

    issue_front(0, 0)
    issue_front(1, 1)
    wait_front(0)
    issue_scatter(0)
    issue_front(2, 2)
    wait_front(1)
    issue_scatter(1)
    wait_scatter(0)
    issue_front(3, 0)
    wait_front(2)
    issue_scatter(2)
    wait_scatter(1)
    issue_front(4, 1)

    @pl.loop(3, NCHUNK - 2, step=NSLOT)
    def _(c):
        for slot_off in range(NSLOT):
            jj = c + slot_off
            slot = (3 + slot_off) % NSLOT
            wait_front(slot)
            issue_scatter(slot)
            nslot = (slot + 2) % NSLOT
            wait_scatter(nslot)
            issue_front(jj + 2, nslot)

    wait_front(0)
    issue_scatter(0)
    wait_front(1)
    issue_scatter(1)
    wait_scatter(2)
    wait_scatter(0)
    wait_scatter(1)

    plsc.subcore_barrier()
    pltpu.sync_copy(acc_sh.at[pl.ds(r0, RPS)], out_hbm.at[cc, pl.ds(r0, RPS)])


_R = 1000


def _tc_matmul_body(x_ref, w_ref, h_ref):
    h_ref[...] = jnp.dot(x_ref[...], w_ref[...],
                         preferred_element_type=jnp.float32)


def _tc_matmul(x, W1):
    return pl.pallas_call(
        _tc_matmul_body,
        grid=(N // _R,),
        in_specs=[
            pl.BlockSpec((_R, F), lambda i: (i, 0)),
            pl.BlockSpec((F, F), lambda i: (0, 0)),
        ],
        out_specs=pl.BlockSpec((_R, F), lambda i: (i, 0)),
        out_shape=jax.ShapeDtypeStruct((N, F), jnp.float32),
    )(x, W1)


def _tc_scale_body(h_ref, c_ref, g_ref):
    g_ref[...] = h_ref[...] * lax.rsqrt(c_ref[...] + 1.0)


def _tc_scale(h, cnt):
    return pl.pallas_call(
        _tc_scale_body,
        grid=(N // _R,),
        in_specs=[
            pl.BlockSpec((_R, F), lambda i: (i, 0)),
            pl.BlockSpec((_R, 1), lambda i: (i, 0)),
        ],
        out_specs=pl.BlockSpec((_R, F), lambda i: (i, 0)),
        out_shape=jax.ShapeDtypeStruct((N, F), jnp.float32),
    )(h, cnt)


def _tc_finish_body(a0_ref, a1_ref, g_ref, c_ref, b1_ref, wc_ref,
                    bc_ref, emb_ref, lsm_ref):
    dinv = lax.rsqrt(c_ref[...] + 1.0)
    s = (a0_ref[0] + a1_ref[0] + g_ref[...]) * dinv + b1_ref[...]
    emb = jnp.maximum(s, 0.0)
    emb_ref[...] = emb
    logits = jnp.dot(emb, wc_ref[...], preferred_element_type=jnp.float32)
    logits = logits + bc_ref[...]
    col = lax.broadcasted_iota(jnp.int32, (_R, F), 1)
    valid = col < FCLS
    lm = jnp.where(valid, logits, jnp.float32(-1e30))
    m = jnp.max(lm, axis=1, keepdims=True)
    ez = jnp.where(valid, jnp.exp(lm - m), 0.0)
    lsm = lm - m - jnp.log(jnp.sum(ez, axis=1, keepdims=True))
    lsm_ref[...] = lsm[:, :FCLS]


def _tc_finish(acc, g, cnt, b1r, wcp, bcp):
    return pl.pallas_call(
        _tc_finish_body,
        grid=(N // _R,),
        in_specs=[
            pl.BlockSpec((1, _R, F), lambda i: (0, i, 0)),
            pl.BlockSpec((1, _R, F), lambda i: (1, i, 0)),
            pl.BlockSpec((_R, F), lambda i: (i, 0)),
            pl.BlockSpec((_R, 1), lambda i: (i, 0)),
            pl.BlockSpec((1, F), lambda i: (0, 0)),
            pl.BlockSpec((F, F), lambda i: (0, 0)),
            pl.BlockSpec((1, F), lambda i: (0, 0)),
        ],
        out_specs=[
            pl.BlockSpec((_R, F), lambda i: (i, 0)),
            pl.BlockSpec((_R, FCLS), lambda i: (i, 0)),
        ],
        out_shape=[
            jax.ShapeDtypeStruct((N, F), jnp.float32),
            jax.ShapeDtypeStruct((N, FCLS), jnp.float32),
        ],
    )(acc, acc, g, cnt, b1r, wcp, bcp)


def kernel(x, adj, W1, b1, Wc, bc):
    zeros128 = jnp.zeros((NPAD, F), jnp.float32)
    zeros1d = jnp.zeros((NPAD,), jnp.float32)
    ones1d = jnp.ones((EPW2,), jnp.float32)

    adjf = adj.reshape(2 * E)

    h = _tc_matmul(x, W1)
    cnt = _sc_degree(adjf, zeros1d, ones1d).reshape(NPAD, 1)
    g = _tc_scale(h, cnt)

    acc = _sc_aggregate(g, adjf, zeros128)

    b1r = b1.reshape(1, F)
    wcp = jnp.pad(Wc, ((0, 0), (0, F - FCLS)))
    bcp = jnp.pad(bc, (0, F - FCLS)).reshape(1, F)

    emb, lsm = _tc_finish(acc, g, cnt, b1r, wcp, bcp)
    return (emb, lsm)


# speedup vs baseline: 1.1434x; 1.0616x over previous
import functools

import jax
import jax.numpy as jnp
from jax import lax
from jax.experimental import pallas as pl
from jax.experimental.pallas import tpu as pltpu
from jax.experimental.pallas import tpu_sc as plsc

N = 10000
E = 320000
F = 128
FCLS = 40

NSC = 2
NSUB = 16
NW = NSC * NSUB
NPAD = 10240
RPS = NPAD // NSUB
EPW = E // NW
K = 200

_sc_mesh = plsc.VectorSubcoreMesh(
    core_axis_name="c", subcore_axis_name="s", num_cores=NSC, num_subcores=NSUB
)


EPW2 = E // NSUB


@functools.partial(
    pl.kernel,
    out_type=jax.ShapeDtypeStruct((NPAD,), jnp.float32),
    mesh=_sc_mesh,
    scratch_types=[
        pltpu.VMEM((EPW2,), jnp.int32),
        pltpu.VMEM((EPW2,), jnp.float32),
        pltpu.VMEM((RPS,), jnp.float32),
        pltpu.VMEM_SHARED((NPAD,), jnp.float32),
        pltpu.SemaphoreType.DMA,
        pltpu.SemaphoreType.DMA,
    ],
)
def _sc_degree(adj_hbm, ones_hbm, out_hbm, idx_v, ones_v, zb, deg_sh, s1, s2):
    cc = lax.axis_index("c")
    sid = lax.axis_index("s")
    r0 = sid * RPS
    base = sid * EPW2
    pltpu.async_copy(adj_hbm.at[pl.ds(E + base, EPW2)], idx_v, s1)
    pltpu.async_copy(ones_hbm, ones_v, s2)

    @pl.loop(0, RPS, step=16)
    def _(i):
        zb[pl.ds(i, 16)] = jnp.zeros((16,), jnp.float32)

    pltpu.sync_copy(zb, deg_sh.at[pl.ds(r0, RPS)])
    pltpu.make_async_copy(adj_hbm.at[pl.ds(E + base, EPW2)], idx_v, s1).wait()
    pltpu.make_async_copy(ones_hbm, ones_v, s2).wait()
    plsc.subcore_barrier()

    pltpu.sync_copy(ones_v, deg_sh.at[idx_v], add=True)

    plsc.subcore_barrier()

    @pl.when(cc == 0)
    def _():
        pltpu.sync_copy(deg_sh.at[pl.ds(r0, RPS)], out_hbm.at[pl.ds(r0, RPS)])


KC = 80
NCHUNK = EPW // KC
NSLOT = 3


@functools.partial(
    pl.kernel,
    out_type=jax.ShapeDtypeStruct((NSC, NPAD, F), jnp.float32),
    mesh=_sc_mesh,
    scratch_types=[
        pltpu.VMEM((EPW,), jnp.int32),
        pltpu.VMEM((NSLOT, 1, KC), jnp.int32),
        pltpu.VMEM((NSLOT, KC, F), jnp.float32),
        pltpu.VMEM_SHARED((NPAD, F), jnp.float32),
        pltpu.SemaphoreType.DMA((NSLOT,)),
        pltpu.SemaphoreType.DMA((NSLOT,)),
        pltpu.SemaphoreType.DMA((NSLOT,)),
        pltpu.SemaphoreType.DMA,
    ],
)
def _sc_aggregate(g_hbm, adj_hbm, out_hbm,
                  srcall, didx, rows_v, acc_sh, sem_i, sem_g, sem_s, sem_m):
    cc = lax.axis_index("c")
    sid = lax.axis_index("s")
    r0 = sid * RPS
    base = (cc * NSUB + sid) * EPW

    pltpu.async_copy(adj_hbm.at[pl.ds(base, EPW)], srcall, sem_m)

    @pl.loop(0, KC, step=1)
    def _(i):
        for c in range(0, F, 16):
            rows_v[0, i, pl.ds(c, 16)] = jnp.zeros((16,), jnp.float32)

    @pl.loop(0, RPS, step=KC)
    def _(i):
        pltpu.sync_copy(rows_v.at[0], acc_sh.at[pl.ds(r0 + i, KC)])

    pltpu.make_async_copy(adj_hbm.at[pl.ds(base, EPW)], srcall, sem_m).wait()
    plsc.subcore_barrier()

    def issue_front(jj, slot):
        pltpu.async_copy(adj_hbm.at[pl.ds(E + base + jj * KC, KC)],
                         didx.at[slot, 0], sem_i.at[slot])
        pltpu.async_copy(g_hbm.at[srcall.at[pl.ds(jj * KC, KC)]],
                         rows_v.at[slot], sem_g.at[slot])

    def wait_front(slot):
        pltpu.make_async_copy(adj_hbm.at[pl.ds(0, KC)], didx.at[slot, 0],
                              sem_i.at[slot]).wait()
        pltpu.make_async_copy(g_hbm.at[pl.ds(0, KC)], rows_v.at[slot],
                              sem_g.at[slot]).wait()

    def issue_scatter(slot):
        pltpu.async_copy(rows_v.at[slot], acc_sh.at[didx.at[slot, 0]],
                         sem_s.at[slot], add=True)

    def wait_scatter(slot):
        pltpu.make_async_copy(g_hbm.at[pl.ds(0, KC)], rows_v.at[slot],
                              sem_s.at[slot]).wait()

    issue_front(0, 0)
    issue_front(1, 1)
    wait_front(0)
    issue_scatter(0)
    issue_front(2, 2)
    wait_front(1)
    issue_scatter(1)
    wait_scatter(0)
    issue_front(3, 0)
    wait_front(2)
    issue_scatter(2)
    wait_scatter(1)
    issue_front(4, 1)

    @pl.loop(3, NCHUNK - 2, step=NSLOT)
    def _(c):
        for slot_off in range(NSLOT):
            jj = c + slot_off
            slot = (3 + slot_off) % NSLOT
            wait_front(slot)
            issue_scatter(slot)
            nslot = (slot + 2) % NSLOT
            wait_scatter(nslot)
            issue_front(jj + 2, nslot)

    wait_front(0)
    issue_scatter(0)
    wait_front(1)
    issue_scatter(1)
    wait_scatter(2)
    wait_scatter(0)
    wait_scatter(1)

    plsc.subcore_barrier()
    pltpu.sync_copy(acc_sh.at[pl.ds(r0, RPS)], out_hbm.at[cc, pl.ds(r0, RPS)])


_R = 2000


def _tc_matmul_body(x_ref, w_ref, h_ref):
    h_ref[...] = jnp.dot(x_ref[...], w_ref[...],
                         preferred_element_type=jnp.float32)


def _tc_matmul(x, W1):
    return pl.pallas_call(
        _tc_matmul_body,
        grid=(N // _R,),
        in_specs=[
            pl.BlockSpec((_R, F), lambda i: (i, 0)),
            pl.BlockSpec((F, F), lambda i: (0, 0)),
        ],
        out_specs=pl.BlockSpec((_R, F), lambda i: (i, 0)),
        out_shape=jax.ShapeDtypeStruct((N, F), jnp.float32),
    )(x, W1)


def _tc_scale_body(h_ref, c_ref, g_ref):
    g_ref[...] = h_ref[...] * lax.rsqrt(c_ref[...] + 1.0)


def _tc_scale(h, cnt):
    return pl.pallas_call(
        _tc_scale_body,
        grid=(N // _R,),
        in_specs=[
            pl.BlockSpec((_R, F), lambda i: (i, 0)),
            pl.BlockSpec((_R, 1), lambda i: (i, 0)),
        ],
        out_specs=pl.BlockSpec((_R, F), lambda i: (i, 0)),
        out_shape=jax.ShapeDtypeStruct((N, F), jnp.float32),
    )(h, cnt)


def _tc_finish_body(a0_ref, a1_ref, g_ref, c_ref, b1_ref, wc_ref,
                    bc_ref, emb_ref, lsm_ref):
    dinv = lax.rsqrt(c_ref[...] + 1.0)
    s = (a0_ref[0] + a1_ref[0] + g_ref[...]) * dinv + b1_ref[...]
    emb = jnp.maximum(s, 0.0)
    emb_ref[...] = emb
    logits = jnp.dot(emb, wc_ref[...], preferred_element_type=jnp.float32)
    logits = logits + bc_ref[...]
    col = lax.broadcasted_iota(jnp.int32, (_R, F), 1)
    valid = col < FCLS
    lm = jnp.where(valid, logits, jnp.float32(-1e30))
    m = jnp.max(lm, axis=1, keepdims=True)
    ez = jnp.where(valid, jnp.exp(lm - m), 0.0)
    lsm = lm - m - jnp.log(jnp.sum(ez, axis=1, keepdims=True))
    lsm_ref[...] = lsm[:, :FCLS]


def _tc_finish(acc, g, cnt, b1r, wcp, bcp):
    return pl.pallas_call(
        _tc_finish_body,
        grid=(N // _R,),
        in_specs=[
            pl.BlockSpec((1, _R, F), lambda i: (0, i, 0)),
            pl.BlockSpec((1, _R, F), lambda i: (1, i, 0)),
            pl.BlockSpec((_R, F), lambda i: (i, 0)),
            pl.BlockSpec((_R, 1), lambda i: (i, 0)),
            pl.BlockSpec((1, F), lambda i: (0, 0)),
            pl.BlockSpec((F, F), lambda i: (0, 0)),
            pl.BlockSpec((1, F), lambda i: (0, 0)),
        ],
        out_specs=[
            pl.BlockSpec((_R, F), lambda i: (i, 0)),
            pl.BlockSpec((_R, FCLS), lambda i: (i, 0)),
        ],
        out_shape=[
            jax.ShapeDtypeStruct((N, F), jnp.float32),
            jax.ShapeDtypeStruct((N, FCLS), jnp.float32),
        ],
    )(acc, acc, g, cnt, b1r, wcp, bcp)


def kernel(x, adj, W1, b1, Wc, bc):
    ones1d = jnp.ones((EPW2,), jnp.float32)
    adjf = adj.reshape(2 * E)

    h = _tc_matmul(x, W1)
    cnt = _sc_degree(adjf, ones1d).reshape(NPAD, 1)
    g = _tc_scale(h, cnt)

    acc = _sc_aggregate(g, adjf)

    b1r = b1.reshape(1, F)
    wcp = jnp.pad(Wc, ((0, 0), (0, F - FCLS)))
    bcp = jnp.pad(bc, (0, F - FCLS)).reshape(1, F)

    emb, lsm = _tc_finish(acc, g, cnt, b1r, wcp, bcp)
    return (emb, lsm)
